# baseline (device time: 105112 ns/iter reference)
import jax
import jax.numpy as jnp
from jax import lax
from jax.experimental import pallas as pl
from jax.experimental.pallas import tpu as pltpu

N_DEV = 16
B = 2
SQ = 256
SKV = 256
HQ = 4
DH = 64
DMODEL = 512
BLK = 64
BH = B * HQ
NP = 4
R_HOPS = 8
L_HOPS = 7


def kernel(x, Wq, K_ext, V_ext, Wo):
    def body(x_ref, wq_ref, k_ref, v_ref, wo_ref, out_ref,
             rbuf, lbuf, r_ss, r_rs, l_ss, l_rs):
        my_pos = lax.axis_index("i")
        left = lax.rem(my_pos - 1 + N_DEV, N_DEV)
        right = lax.rem(my_pos + 1, N_DEV)

        barrier_sem = pltpu.get_barrier_semaphore()
        for nbr in (left, right):
            pl.semaphore_signal(
                barrier_sem, inc=1,
                device_id=(nbr,), device_id_type=pl.DeviceIdType.MESH,
            )
        pl.semaphore_wait(barrier_sem, 2)

        kt = jnp.transpose(
            k_ref[...].astype(jnp.bfloat16), (0, 2, 1, 3)).reshape(
                2, BH // 2, SKV, DH)
        vt = jnp.transpose(
            v_ref[...].astype(jnp.bfloat16), (0, 2, 1, 3)).reshape(
                2, BH // 2, SKV, DH)
        rbuf[0, 0:2] = kt
        rbuf[0, 2:4] = vt
        lbuf[0, 0:2] = kt
        lbuf[0, 2:4] = vt

        def pieces(buf, ssems, rsems, h, dst):
            return [
                pltpu.make_async_remote_copy(
                    src_ref=buf.at[h, p],
                    dst_ref=buf.at[h + 1, p],
                    send_sem=ssems.at[h, p],
                    recv_sem=rsems.at[h, p],
                    device_id=(dst,),
                    device_id_type=pl.DeviceIdType.MESH,
                )
                for p in range(NP)
            ]

        wq = wq_ref[...].astype(jnp.bfloat16)
        q_all = jnp.stack([
            jnp.transpose(
                (jnp.dot(x_ref[b].astype(jnp.bfloat16), wq,
                         preferred_element_type=jnp.float32) * 0.125)
                .astype(jnp.bfloat16).reshape(SQ, HQ, DH),
                (1, 0, 2))
            for b in range(B)
        ]).reshape(BH, SQ, DH)

        li = lax.broadcasted_iota(jnp.int32, (SQ, SKV), 0)
        lj = lax.broadcasted_iota(jnp.int32, (SQ, SKV), 1)
        diag_bias = jnp.where((lj // BLK) <= (li // BLK),
                              0.0, -1e9).astype(jnp.float32)

        def fold(buf, s, num, den, bias):
            k_c = buf[s, 0:2].reshape(BH, SKV, DH)
            v_c = buf[s, 2:4].reshape(BH, SKV, DH)
            scores = lax.dot_general(
                q_all, k_c,
                dimension_numbers=(((2,), (2,)), ((0,), (0,))),
                preferred_element_type=jnp.float32,
            )
            w = jnp.exp(scores + bias)
            num = num + lax.dot_general(
                w.astype(jnp.bfloat16), v_c,
                dimension_numbers=(((2,), (1,)), ((0,), (0,))),
                preferred_element_type=jnp.float32,
            )
            den = den + jnp.sum(w, axis=-1, keepdims=True)
            return num, den

        r_ops = pieces(rbuf, r_ss, r_rs, 0, right)
        l_ops = pieces(lbuf, l_ss, l_rs, 0, left)
        for r in r_ops + l_ops:
            r.start()
        num = jnp.zeros((BH, SQ, DH), jnp.float32)
        den = jnp.zeros((BH, SQ, 1), jnp.float32)
        num, den = fold(rbuf, 0, num, den, diag_bias[None])

        prev_r, prev_l = r_ops, l_ops
        for s in range(1, R_HOPS + 1):
            nr = pieces(rbuf, r_ss, r_rs, s, right) if s < R_HOPS else None
            nl = pieces(lbuf, l_ss, l_rs, s, left) if s < L_HOPS else None
            for p in range(NP):
                prev_r[p].wait_recv()
                if nr:
                    nr[p].start()
                if s <= L_HOPS:
                    prev_l[p].wait_recv()
                    if nl:
                        nl[p].start()
            r_bias = jnp.where(s <= my_pos, 0.0, -1e9).astype(jnp.float32)
            num, den = fold(rbuf, s, num, den, r_bias)
            if s <= L_HOPS:
                l_bias = jnp.where(my_pos + s >= N_DEV, 0.0,
                                   -1e9).astype(jnp.float32)
                num, den = fold(lbuf, s, num, den, l_bias)
            for r in prev_r:
                r.wait_send()
            if s <= L_HOPS:
                for r in prev_l:
                    r.wait_send()
            prev_r, prev_l = nr, nl

        ctx = (num / den).reshape(B, HQ, SQ, DH)
        wo = wo_ref[...].astype(jnp.bfloat16)
        for b in range(B):
            ctx_b = jnp.transpose(ctx[b], (1, 0, 2)).reshape(SQ, HQ * DH)
            out_ref[b] = jnp.dot(ctx_b.astype(jnp.bfloat16), wo,
                                 preferred_element_type=jnp.float32)

    return pl.pallas_call(
        body,
        out_shape=jax.ShapeDtypeStruct((B, SQ, DMODEL), jnp.float32),
        in_specs=[pl.BlockSpec(memory_space=pltpu.VMEM)] * 5,
        out_specs=pl.BlockSpec(memory_space=pltpu.VMEM),
        scratch_shapes=[
            pltpu.VMEM((R_HOPS + 1, NP, BH // 2, SKV, DH), jnp.bfloat16),
            pltpu.VMEM((L_HOPS + 1, NP, BH // 2, SKV, DH), jnp.bfloat16),
            pltpu.SemaphoreType.DMA((R_HOPS, NP)),
            pltpu.SemaphoreType.DMA((R_HOPS, NP)),
            pltpu.SemaphoreType.DMA((L_HOPS, NP)),
            pltpu.SemaphoreType.DMA((L_HOPS, NP)),
        ],
        compiler_params=pltpu.CompilerParams(collective_id=0),
    )(x, Wq, K_ext, V_ext, Wo)


# device time: 104538 ns/iter; 1.0055x vs baseline; 1.0055x over previous
import jax
import jax.numpy as jnp
from jax import lax
from jax.experimental import pallas as pl
from jax.experimental.pallas import tpu as pltpu

N_DEV = 16
B = 2
SQ = 256
SKV = 256
HQ = 4
DH = 64
DMODEL = 512
BLK = 64
BH = B * HQ
R_HOPS = 8
L_HOPS = 7


def kernel(x, Wq, K_ext, V_ext, Wo):
    def body(x_ref, wq_ref, k_ref, v_ref, wo_ref, out_ref,
             rbuf, lbuf,
             rk_ss, rk_rs, rv_ss, rv_rs,
             lk_ss, lk_rs, lv_ss, lv_rs):
        my_pos = lax.axis_index("i")
        left = lax.rem(my_pos - 1 + N_DEV, N_DEV)
        right = lax.rem(my_pos + 1, N_DEV)

        barrier_sem = pltpu.get_barrier_semaphore()
        for nbr in (left, right):
            pl.semaphore_signal(
                barrier_sem, inc=1,
                device_id=(nbr,), device_id_type=pl.DeviceIdType.MESH,
            )
        pl.semaphore_wait(barrier_sem, 2)

        kt = jnp.transpose(
            k_ref[...].astype(jnp.bfloat16), (0, 2, 1, 3)).reshape(
                BH, SKV, DH)
        vt = jnp.transpose(
            v_ref[...].astype(jnp.bfloat16), (0, 2, 1, 3)).reshape(
                BH, SKV, DH)
        rbuf[0, 0] = kt
        rbuf[0, 1] = vt
        lbuf[0, 0] = kt
        lbuf[0, 1] = vt

        def piece(buf, ssems, rsems, h, part, dst):
            return pltpu.make_async_remote_copy(
                src_ref=buf.at[h, part],
                dst_ref=buf.at[h + 1, part],
                send_sem=ssems.at[h],
                recv_sem=rsems.at[h],
                device_id=(dst,),
                device_id_type=pl.DeviceIdType.MESH,
            )

        def r_pieces(h):
            return (piece(rbuf, rk_ss, rk_rs, h, 0, right),
                    piece(rbuf, rv_ss, rv_rs, h, 1, right))

        def l_pieces(h):
            return (piece(lbuf, lk_ss, lk_rs, h, 0, left),
                    piece(lbuf, lv_ss, lv_rs, h, 1, left))

        wq = wq_ref[...].astype(jnp.bfloat16)
        q_all = jnp.stack([
            jnp.transpose(
                (jnp.dot(x_ref[b].astype(jnp.bfloat16), wq,
                         preferred_element_type=jnp.float32) * 0.125)
                .astype(jnp.bfloat16).reshape(SQ, HQ, DH),
                (1, 0, 2))
            for b in range(B)
        ]).reshape(BH, SQ, DH)

        li = lax.broadcasted_iota(jnp.int32, (SQ, SKV), 0)
        lj = lax.broadcasted_iota(jnp.int32, (SQ, SKV), 1)
        diag_bias = jnp.where((lj // BLK) <= (li // BLK),
                              0.0, -1e9).astype(jnp.float32)

        def fold(buf, s, num, den, bias):
            k_c = buf[s, 0]
            v_c = buf[s, 1]
            scores = lax.dot_general(
                q_all, k_c,
                dimension_numbers=(((2,), (2,)), ((0,), (0,))),
                preferred_element_type=jnp.float32,
            )
            w = jnp.exp(scores + bias)
            num = num + lax.dot_general(
                w.astype(jnp.bfloat16), v_c,
                dimension_numbers=(((2,), (1,)), ((0,), (0,))),
                preferred_element_type=jnp.float32,
            )
            den = den + jnp.sum(w, axis=-1, keepdims=True)
            return num, den

        r_ops = r_pieces(0)
        l_ops = l_pieces(0)
        for r in r_ops + l_ops:
            r.start()
        num = jnp.zeros((BH, SQ, DH), jnp.float32)
        den = jnp.zeros((BH, SQ, 1), jnp.float32)
        num, den = fold(rbuf, 0, num, den, diag_bias[None])

        prev_r, prev_l = r_ops, l_ops
        for s in range(1, R_HOPS + 1):
            nr = r_pieces(s) if s < R_HOPS else None
            nl = l_pieces(s) if s < L_HOPS else None
            prev_r[0].wait_recv()
            if nr:
                nr[0].start()
            if s <= L_HOPS:
                prev_l[0].wait_recv()
                if nl:
                    nl[0].start()
            prev_r[1].wait_recv()
            if nr:
                nr[1].start()
            if s <= L_HOPS:
                prev_l[1].wait_recv()
                if nl:
                    nl[1].start()
            r_bias = jnp.where(s <= my_pos, 0.0, -1e9).astype(jnp.float32)
            num, den = fold(rbuf, s, num, den, r_bias)
            if s <= L_HOPS:
                l_bias = jnp.where(my_pos + s >= N_DEV, 0.0,
                                   -1e9).astype(jnp.float32)
                num, den = fold(lbuf, s, num, den, l_bias)
            for r in prev_r:
                r.wait_send()
            if s <= L_HOPS:
                for r in prev_l:
                    r.wait_send()
            prev_r, prev_l = nr, nl

        ctx = (num / den).reshape(B, HQ, SQ, DH)
        wo = wo_ref[...].astype(jnp.bfloat16)
        for b in range(B):
            ctx_b = jnp.transpose(ctx[b], (1, 0, 2)).reshape(SQ, HQ * DH)
            out_ref[b] = jnp.dot(ctx_b.astype(jnp.bfloat16), wo,
                                 preferred_element_type=jnp.float32)

    return pl.pallas_call(
        body,
        out_shape=jax.ShapeDtypeStruct((B, SQ, DMODEL), jnp.float32),
        in_specs=[pl.BlockSpec(memory_space=pltpu.VMEM)] * 5,
        out_specs=pl.BlockSpec(memory_space=pltpu.VMEM),
        scratch_shapes=[
            pltpu.VMEM((R_HOPS + 1, 2, BH, SKV, DH), jnp.bfloat16),
            pltpu.VMEM((L_HOPS + 1, 2, BH, SKV, DH), jnp.bfloat16),
            pltpu.SemaphoreType.DMA((R_HOPS,)),
            pltpu.SemaphoreType.DMA((R_HOPS,)),
            pltpu.SemaphoreType.DMA((R_HOPS,)),
            pltpu.SemaphoreType.DMA((R_HOPS,)),
            pltpu.SemaphoreType.DMA((L_HOPS,)),
            pltpu.SemaphoreType.DMA((L_HOPS,)),
            pltpu.SemaphoreType.DMA((L_HOPS,)),
            pltpu.SemaphoreType.DMA((L_HOPS,)),
        ],
        compiler_params=pltpu.CompilerParams(collective_id=0),
    )(x, Wq, K_ext, V_ext, Wo)
